# baseline (speedup 1.0000x reference)
import jax
import jax.numpy as jnp
from jax.experimental import pallas as pl


def _expand_tc(table, batch_size):
    nqueries, d = table.shape
    bq = 150
    t3 = table.reshape(nqueries // bq, bq, d)

    def body(tab_ref, out_ref):
        out_ref[...] = jnp.broadcast_to(
            tab_ref[0][:, None, :], (bq, batch_size, d)
        )

    return pl.pallas_call(
        body,
        grid=(nqueries // bq,),
        in_specs=[pl.BlockSpec((1, bq, d), lambda i: (i, 0, 0))],
        out_specs=pl.BlockSpec((bq, batch_size, d), lambda i: (i, 0, 0)),
        out_shape=jax.ShapeDtypeStruct(
            (nqueries, batch_size, d), jnp.float32
        ),
    )(t3)


def kernel(batch_ref, table):
    out_t = _expand_tc(table, batch_ref.shape[0])
    return jnp.transpose(out_t, (1, 0, 2))


# speedup vs baseline: 1.0248x; 1.0248x over previous
import jax
import jax.numpy as jnp
from jax.experimental import pallas as pl


def _expand_tc(table, batch_size):
    nqueries, d = table.shape
    bq = 60
    t3 = table.reshape(nqueries // bq, bq, d)

    def body(tab_ref, out_ref):
        out_ref[...] = jnp.broadcast_to(
            tab_ref[0][:, None, :], (bq, batch_size, d)
        )

    return pl.pallas_call(
        body,
        grid=(nqueries // bq,),
        in_specs=[pl.BlockSpec((1, bq, d), lambda i: (i, 0, 0))],
        out_specs=pl.BlockSpec((bq, batch_size, d), lambda i: (i, 0, 0)),
        out_shape=jax.ShapeDtypeStruct(
            (nqueries, batch_size, d), jnp.float32
        ),
    )(t3)


def kernel(batch_ref, table):
    out_t = _expand_tc(table, batch_ref.shape[0])
    return jnp.transpose(out_t, (1, 0, 2))
